# baseline (speedup 1.0000x reference)
import functools

import jax
import jax.numpy as jnp
from jax import lax
from jax.experimental import pallas as pl
from jax.experimental.pallas import tpu as pltpu
from jax.experimental.pallas import tpu_sc as plsc

VOCAB = 1_000_000
EMBED_DIM = 32
BATCH = 16384
HIST = 50

NUM_WORKERS = 32
TOTAL = BATCH * HIST
STRIP = BATCH // NUM_WORKERS
GATHER = 128
CHUNKS_PER_BLK = STRIP // GATHER
NUM_BLOCKS = HIST

_mesh = plsc.VectorSubcoreMesh(core_axis_name="c", subcore_axis_name="s")


@functools.partial(
    pl.kernel,
    mesh=_mesh,
    out_type=jax.ShapeDtypeStruct((HIST, BATCH, EMBED_DIM), jnp.float32),
    compiler_params=pltpu.CompilerParams(use_tc_tiling_on_sc=False),
    scratch_types=[
        pltpu.VMEM((NUM_BLOCKS, STRIP), jnp.int32),
        pltpu.VMEM((STRIP, EMBED_DIM), jnp.float32),
        pltpu.VMEM((STRIP, EMBED_DIM), jnp.float32),
        pltpu.SemaphoreType.DMA,
        [pltpu.SemaphoreType.DMA] * 2,
        [pltpu.SemaphoreType.DMA] * 2,
    ],
)
def _embed_gather(x_hbm, table_hbm, out_hbm, idx_v, buf0, buf1, semi, semg,
                  semw):
    wid = lax.axis_index("s") * 2 + lax.axis_index("c")
    b0 = wid * STRIP
    bufs = (buf0, buf1)

    stages = [
        pltpu.async_copy(x_hbm.at[pl.ds(h * BATCH + b0, STRIP)],
                         idx_v.at[h], semi)
        for h in range(NUM_BLOCKS)
    ]
    for d in stages:
        d.wait()

    def gathers(h, p, start=True):
        mk = pltpu.async_copy if start else pltpu.make_async_copy
        return [
            mk(table_hbm.at[idx_v.at[h, pl.ds(c * GATHER, GATHER)]],
               bufs[p].at[pl.ds(c * GATHER, GATHER)],
               semg[p])
            for c in range(CHUNKS_PER_BLK)
        ]

    def writeback(h, p, start=True):
        mk = pltpu.async_copy if start else pltpu.make_async_copy
        return mk(bufs[p], out_hbm.at[h, pl.ds(b0, STRIP)], semw[p])

    gathers(0, 0)
    gathers(1, 1)

    def body(pi, carry):
        h0 = 2 * pi
        for d in gathers(h0, 0, start=False):
            d.wait()
        writeback(h0, 0)

        for d in gathers(h0 + 1, 1, start=False):
            d.wait()

        writeback(h0, 0, start=False).wait()
        gathers(h0 + 2, 0)

        writeback(h0 + 1, 1)
        writeback(h0 + 1, 1, start=False).wait()
        gathers(h0 + 3, 1)
        return carry

    lax.fori_loop(0, NUM_BLOCKS // 2 - 1, body, 0)

    for d in gathers(NUM_BLOCKS - 2, 0, start=False):
        d.wait()
    writeback(NUM_BLOCKS - 2, 0)
    for d in gathers(NUM_BLOCKS - 1, 1, start=False):
        d.wait()
    writeback(NUM_BLOCKS - 1, 1)
    writeback(NUM_BLOCKS - 2, 0, start=False).wait()
    writeback(NUM_BLOCKS - 1, 1, start=False).wait()


def kernel(x, table):
    idx = x.T.reshape(TOTAL)
    out = _embed_gather(idx, table)
    return out.transpose(1, 0, 2)
